# Initial kernel scaffold; baseline (speedup 1.0000x reference)
import jax
import jax.numpy as jnp
from jax.experimental import pallas as pl


def kernel(feats, gt_prob, fov_mask):
    raise NotImplementedError("write your pallas kernel here")


# speedup vs baseline: 3.7784x; 3.7784x over previous
import functools

import jax
import jax.numpy as jnp
from jax.experimental import pallas as pl
from jax.experimental.pallas import tpu as pltpu

_VIEWS = 1
_TEMP = 0.4
_IGNORE = 0
_MAXPPC = 150
_CLIP_POS = 1.0
_WEIGHT = 1.0
_BLK = 512


def _supcon_body(s_ref, labr_ref, labc_ref, f_ref, out_ref, fn_ref, d_ref):
    S = s_ref[0]
    nb = (S + _BLK - 1) // _BLK
    inv_t = 1.0 / _TEMP
    shift = 1.0 / _TEMP

    def norm_body(rb, _):
        blk = f_ref[pl.ds(rb * _BLK, _BLK), :]
        n = jnp.sqrt(jnp.sum(blk * blk, axis=1, keepdims=True))
        fn_ref[pl.ds(rb * _BLK, _BLK), :] = blk / (n + 1e-12)
        return 0

    jax.lax.fori_loop(0, nb, norm_body, 0)

    def d_body(rb, _):
        labr = labr_ref[pl.ds(rb * _BLK, _BLK), :]
        a = fn_ref[pl.ds(rb * _BLK, _BLK), :]

        def cb_body(cb, acc):
            b = fn_ref[pl.ds(cb * _BLK, _BLK), :]
            s = jax.lax.dot_general(a, b, (((1,), (1,)), ((), ())),
                                    preferred_element_type=jnp.float32)
            l = s * inv_t - shift
            labc = labc_ref[:, pl.ds(cb * _BLK, _BLK)]
            colidx = jax.lax.broadcasted_iota(jnp.int32, (1, _BLK), 1) + cb * _BLK
            negm = (labr != labc) & (colidx < S)
            return acc + jnp.sum(jnp.where(negm, jnp.exp(l), 0.0), axis=1,
                                 keepdims=True)

        acc = jax.lax.fori_loop(0, nb, cb_body,
                                jnp.zeros((_BLK, 1), jnp.float32))
        d_ref[pl.ds(rb * _BLK, _BLK), :] = acc
        return 0

    jax.lax.fori_loop(0, nb, d_body, 0)

    def p_body(rb, carry):
        tot_p, tot_c = carry
        labr = labr_ref[pl.ds(rb * _BLK, _BLK), :]
        a = fn_ref[pl.ds(rb * _BLK, _BLK), :]
        dvec = d_ref[pl.ds(rb * _BLK, _BLK), :]
        rowidx = jax.lax.broadcasted_iota(jnp.int32, (_BLK, 1), 0) + rb * _BLK

        def cb_body(cb, carry2):
            psum, pcnt = carry2
            b = fn_ref[pl.ds(cb * _BLK, _BLK), :]
            s = jax.lax.dot_general(a, b, (((1,), (1,)), ((), ())),
                                    preferred_element_type=jnp.float32)
            labc = labc_ref[:, pl.ds(cb * _BLK, _BLK)]
            colidx = jax.lax.broadcasted_iota(jnp.int32, (_BLK, _BLK), 1) + cb * _BLK
            posm = (labr == labc) & (rowidx != colidx) & (colidx < S)
            l = jnp.minimum(s, _CLIP_POS) * inv_t - shift
            lp = l - jnp.log(jnp.exp(l) + dvec + 1e-12)
            psum = psum + jnp.sum(jnp.where(posm, lp, 0.0), axis=1,
                                  keepdims=True)
            pcnt = pcnt + jnp.sum(posm.astype(jnp.int32), axis=1,
                                  keepdims=True)
            return psum, pcnt

        psum, pcnt = jax.lax.fori_loop(
            0, nb, cb_body,
            (jnp.zeros((_BLK, 1), jnp.float32),
             jnp.zeros((_BLK, 1), jnp.int32)))
        haspos = pcnt > 0
        mlpp = psum / jnp.maximum(pcnt, 1).astype(jnp.float32)
        tot_p = tot_p + jnp.sum(jnp.where(haspos, mlpp, 0.0))
        tot_c = tot_c + jnp.sum(haspos.astype(jnp.int32))
        return tot_p, tot_c

    tot_p, tot_c = jax.lax.fori_loop(0, nb, p_body,
                                     (jnp.float32(0.0), jnp.int32(0)))
    loss = -tot_p / jnp.maximum(tot_c, 1).astype(jnp.float32)
    out_ref[0, 0] = _WEIGHT * loss


def _supcon_loss(f_sel, lab_sel, S, M):
    labr = lab_sel.reshape(M, 1)
    labc = lab_sel.reshape(1, M)
    out = pl.pallas_call(
        _supcon_body,
        out_shape=jax.ShapeDtypeStruct((1, 1), jnp.float32),
        in_specs=[
            pl.BlockSpec(memory_space=pltpu.SMEM),
            pl.BlockSpec(memory_space=pltpu.VMEM),
            pl.BlockSpec(memory_space=pltpu.VMEM),
            pl.BlockSpec(memory_space=pltpu.VMEM),
        ],
        out_specs=pl.BlockSpec(memory_space=pltpu.SMEM),
        scratch_shapes=[
            pltpu.VMEM((M, 128), jnp.float32),
            pltpu.VMEM((M, 1), jnp.float32),
        ],
    )(S.reshape(1), labr, labc, f_sel)
    return out[0, 0]


def _select_compact(feats, gt_prob, fov_mask):
    BV, Z, H, W = feats.shape
    B = BV // _VIEWS
    C = gt_prob.shape[1]
    M = B * H * W
    gt_label = jnp.argmax(gt_prob, axis=1)
    valid = fov_mask & (gt_label != _IGNORE)
    valid_flat = valid.reshape(M)
    lab_flat = gt_label.reshape(M)
    counts = jnp.zeros((C,), jnp.int32).at[lab_flat].add(
        valid_flat.astype(jnp.int32))
    k = jnp.sum(counts > 0)
    s = jnp.sort(counts)
    lo = s[C - k + (k - 1) // 2]
    hi = s[C - k + k // 2]
    median = (lo + hi) / 2.0
    mean_count = jnp.maximum(jnp.floor(median).astype(jnp.int32), _MAXPPC)
    onehot = (lab_flat[:, None] == jnp.arange(C)[None, :]) & valid_flat[:, None]
    rank = jnp.cumsum(onehot.astype(jnp.int32), axis=0)[jnp.arange(M), lab_flat] - 1
    sel = valid_flat & (rank < mean_count)
    sel_count = jnp.minimum(counts, mean_count)
    seg_start = jnp.concatenate(
        [jnp.zeros((1,), jnp.int32), jnp.cumsum(sel_count)])
    S = seg_start[C]
    dst = seg_start[lab_flat] + rank
    dst = jnp.where(sel, dst, M)
    f_flat = feats.transpose(0, 2, 3, 1).reshape(M, Z)
    f_sel = jnp.zeros((M, Z), jnp.float32).at[dst].set(f_flat, mode='drop')
    lab_sel = jnp.full((M,), -1, jnp.int32).at[dst].set(
        lab_flat.astype(jnp.int32), mode='drop')
    return f_sel, lab_sel, S.astype(jnp.int32), M


def kernel(feats, gt_prob, fov_mask):
    f_sel, lab_sel, S, M = _select_compact(feats, gt_prob, fov_mask)
    return _supcon_loss(f_sel, lab_sel, S, M)
